# Initial kernel scaffold; baseline (speedup 1.0000x reference)
import jax
import jax.numpy as jnp
from jax.experimental import pallas as pl


def kernel(inputs, embedding):
    raise NotImplementedError("write your pallas kernel here")


# speedup vs baseline: 1.5138x; 1.5138x over previous
import functools

import jax
import jax.numpy as jnp
from jax import lax
from jax.experimental import pallas as pl
from jax.experimental.pallas import tpu as pltpu
from jax.experimental.pallas import tpu_sc as plsc

_NC = 2
_NS = 16
_NW = _NC * _NS

_CH = 128
_K = 8
_NBUF = 2


def kernel(inputs, embedding):
    B, H = inputs.shape
    V, D = embedding.shape
    N = B * H
    assert N % (_NW * _CH * _K) == 0
    n_per_w = N // _NW
    n_ch = n_per_w // _CH
    n_grp = n_ch // _K
    assert n_grp % _NBUF == 0

    idx = inputs.reshape(_NW, n_ch, _CH)
    mesh = plsc.VectorSubcoreMesh(core_axis_name="c", subcore_axis_name="s")

    @functools.partial(
        pl.kernel,
        out_type=jax.ShapeDtypeStruct((N, D), jnp.float32),
        mesh=mesh,
        compiler_params=pltpu.CompilerParams(use_tc_tiling_on_sc=False),
        scratch_types=[
            pltpu.VMEM((n_ch, _CH), jnp.int32),
            pltpu.VMEM((_NBUF, _K * _CH, D), jnp.float32),
            pltpu.SemaphoreType.DMA,
        ],
    )
    def _embed(idx_hbm, tab_hbm, out_hbm, idx_v, rows_v, gsem):
        wid = lax.axis_index("s") * _NC + lax.axis_index("c")
        base = wid * n_per_w
        pltpu.sync_copy(idx_hbm.at[wid], idx_v)

        def fire(grp, buf):
            for t in range(_K):
                ch = grp * _K + t
                pltpu.async_copy(
                    tab_hbm.at[idx_v.at[ch]],
                    rows_v.at[buf, pl.ds(t * _CH, _CH)],
                    gsem,
                )

        def drain(grp, buf):
            for t in range(_K):
                ch = grp * _K + t
                pltpu.make_async_copy(
                    tab_hbm.at[idx_v.at[ch]],
                    rows_v.at[buf, pl.ds(t * _CH, _CH)],
                    gsem,
                ).wait()

        fire(0, 0)

        @pl.loop(0, n_grp, step=_NBUF)
        def _grp_loop(g0):
            for b in range(_NBUF):
                g = g0 + b

                @pl.when(g + 1 < n_grp)
                def _():
                    fire(g + 1, (b + 1) % _NBUF)

                drain(g, b)
                pltpu.sync_copy(
                    rows_v.at[b],
                    out_hbm.at[pl.ds(base + g * _K * _CH, _K * _CH)],
                )

    out = _embed(idx, embedding)
    return out.reshape(B, H, D)


# speedup vs baseline: 1.6077x; 1.0620x over previous
import functools

import jax
import jax.numpy as jnp
from jax import lax
from jax.experimental import pallas as pl
from jax.experimental.pallas import tpu as pltpu
from jax.experimental.pallas import tpu_sc as plsc

_NC = 2
_NS = 16
_NW = _NC * _NS

_CH = 128
_K = 8
_NBUF = 2


def kernel(inputs, embedding):
    B, H = inputs.shape
    V, D = embedding.shape
    N = B * H
    assert N % (_NW * _CH * _K) == 0
    n_per_w = N // _NW
    n_ch = n_per_w // _CH
    n_grp = n_ch // _K
    assert n_grp % _NBUF == 0

    idx = inputs.T.reshape(_NW, n_ch, _CH)
    mesh = plsc.VectorSubcoreMesh(core_axis_name="c", subcore_axis_name="s")

    @functools.partial(
        pl.kernel,
        out_type=jax.ShapeDtypeStruct((N, D), jnp.float32),
        mesh=mesh,
        compiler_params=pltpu.CompilerParams(use_tc_tiling_on_sc=False),
        scratch_types=[
            pltpu.VMEM((n_ch, _CH), jnp.int32),
            pltpu.VMEM((_NBUF, _K * _CH, D), jnp.float32),
            pltpu.SemaphoreType.DMA,
        ],
    )
    def _embed(idx_hbm, tab_hbm, out_hbm, idx_v, rows_v, gsem):
        wid = lax.axis_index("s") * _NC + lax.axis_index("c")
        base = wid * n_per_w
        pltpu.sync_copy(idx_hbm.at[wid], idx_v)

        def fire(grp, buf):
            for t in range(_K):
                ch = grp * _K + t
                pltpu.async_copy(
                    tab_hbm.at[idx_v.at[ch]],
                    rows_v.at[buf, pl.ds(t * _CH, _CH)],
                    gsem,
                )

        def drain(grp, buf):
            for t in range(_K):
                ch = grp * _K + t
                pltpu.make_async_copy(
                    tab_hbm.at[idx_v.at[ch]],
                    rows_v.at[buf, pl.ds(t * _CH, _CH)],
                    gsem,
                ).wait()

        fire(0, 0)

        @pl.loop(0, n_grp, step=_NBUF)
        def _grp_loop(g0):
            for b in range(_NBUF):
                g = g0 + b

                @pl.when(g + 1 < n_grp)
                def _():
                    fire(g + 1, (b + 1) % _NBUF)

                drain(g, b)
                pltpu.sync_copy(
                    rows_v.at[b],
                    out_hbm.at[pl.ds(base + g * _K * _CH, _K * _CH)],
                )

    out = _embed(idx, embedding)
    return out.reshape(H, B, D).transpose(1, 0, 2)
